# Initial kernel scaffold; baseline (speedup 1.0000x reference)
import jax
import jax.numpy as jnp
from jax.experimental import pallas as pl


def kernel(input_ids, hash_tables, pos_table, tt_table, gamma, beta):
    raise NotImplementedError("write your pallas kernel here")


# speedup vs baseline: 1.2679x; 1.2679x over previous
import functools

import jax
import jax.numpy as jnp
from jax import lax
from jax.experimental import pallas as pl
from jax.experimental.pallas import tpu as pltpu
from jax.experimental.pallas import tpu_sc as plsc

_HASH_PRIMES = (31, 43, 59, 61, 73, 97, 103, 113)
_L = 16


def _rsqrt16(x):
    i = lax.bitcast_convert_type(x, jnp.int32)
    y = lax.bitcast_convert_type(jnp.int32(0x5F3759DF) - (i >> 1), jnp.float32)
    for _ in range(3):
        y = y * (1.5 - 0.5 * x * y * y)
    return y


@functools.lru_cache(maxsize=None)
def _make_sc_kernel(B, S, NH, NB, SH):
    H = NH * SH
    NC, NS = 2, 16
    NW = NC * NS
    assert S % NW == 0
    NSW = S // NW
    CS = 16
    NCH = NSW // CS
    TPC = B * CS
    RPC = TPC * NH
    NG = RPC // 128
    KV = H // _L
    VPH = TPC // _L
    assert SH % _L == 0 and RPC % 128 == 0 and TPC % _L == 0

    mesh = plsc.VectorSubcoreMesh(core_axis_name="c", subcore_axis_name="s")

    @functools.partial(
        pl.kernel,
        out_type=jax.ShapeDtypeStruct((B * S * H,), jnp.float32),
        mesh=mesh,
        compiler_params=pltpu.CompilerParams(use_tc_tiling_on_sc=False),
        scratch_types=[
            pltpu.VMEM((TPC,), jnp.int32),
            pltpu.VMEM((NG, 128), jnp.int32),
            pltpu.VMEM((RPC, SH), jnp.float32),
            pltpu.VMEM((TPC * H,), jnp.float32),
            pltpu.VMEM((CS * H,), jnp.float32),
            pltpu.VMEM((H,), jnp.float32),
            pltpu.VMEM((2 * H,), jnp.float32),
            pltpu.VMEM((6 * _L,), jnp.float32),
            pltpu.SemaphoreType.DMA,
        ],
    )
    def k(ids_hbm, tab_hbm, pos_hbm, tt_hbm, gamma_hbm, beta_hbm,
          out_hbm, ids_v, idx_v, rows_v, out_v, pos_v, tt_v, gb_v, red_v, sem):
        wid = lax.axis_index("s") * NC + lax.axis_index("c")
        zeros = jnp.zeros((_L,), jnp.float32)
        for z in range(6):
            red_v[pl.ds(z * _L, _L)] = zeros
        pltpu.sync_copy(tt_hbm, tt_v)
        pltpu.sync_copy(gamma_hbm, gb_v.at[pl.ds(0, H)])
        pltpu.sync_copy(beta_hbm, gb_v.at[pl.ds(H, H)])
        s_base = wid * NSW
        lane = lax.iota(jnp.int32, _L)
        bmask = {sh: (lane & sh) == 0 for sh in (8, 4, 2, 1)}

        def chunk_body(ci, _):
            s0 = s_base + ci * CS
            pltpu.sync_copy(pos_hbm.at[pl.ds(s0 * H, CS * H)], pos_v)
            for b in range(B):
                pltpu.sync_copy(ids_hbm.at[pl.ds(b * S + s0, CS)],
                                ids_v.at[pl.ds(b * CS, CS)])

            def tt_body(j, _):
                for kk in range(KV):
                    off = j * H + kk * _L
                    pos_v[pl.ds(off, _L)] = (pos_v[pl.ds(off, _L)]
                                             + tt_v[pl.ds(kk * _L, _L)])
                return 0
            lax.fori_loop(0, CS, tt_body, 0)

            for i in range(NH):
                for j in range(VPH):
                    ids16 = ids_v[pl.ds(j * _L, _L)]
                    h = ((ids16 + 1) * _HASH_PRIMES[i]) & (NB - 1)
                    f = i * TPC + j * _L
                    idx_v[f // 128, pl.ds(f % 128, _L)] = h + i * NB

            descs = [
                pltpu.async_copy(tab_hbm.at[idx_v.at[q]],
                                 rows_v.at[pl.ds(q * 128, 128), :], sem)
                for q in range(NG)
            ]
            for dsc in descs:
                dsc.wait()

            def tok_body(t, _):
                scp = t & (CS - 1)
                acc = jnp.zeros((_L,), jnp.float32)
                acc2 = jnp.zeros((_L,), jnp.float32)
                for kk in range(KV):
                    i, col = divmod(kk * _L, SH)
                    v = rows_v[i * TPC + t, pl.ds(col, _L)]
                    v = v + pos_v[pl.ds(scp * H + kk * _L, _L)]
                    out_v[pl.ds(t * H + kk * _L, _L)] = v
                    acc = acc + v
                    acc2 = acc2 + v * v
                for sh in (8, 4, 2, 1):
                    red_v[pl.ds(_L, _L)] = acc
                    red_v[pl.ds(4 * _L, _L)] = acc2
                    acc = acc + jnp.where(bmask[sh],
                                          red_v[pl.ds(_L + sh, _L)],
                                          red_v[pl.ds(_L - sh, _L)])
                    acc2 = acc2 + jnp.where(bmask[sh],
                                            red_v[pl.ds(4 * _L + sh, _L)],
                                            red_v[pl.ds(4 * _L - sh, _L)])
                mv = acc * (1.0 / H)
                var = acc2 * (1.0 / H) - mv * mv
                rstd = _rsqrt16(var + 1e-12)
                for kk in range(KV):
                    off = t * H + kk * _L
                    v = out_v[pl.ds(off, _L)]
                    y = ((v - mv) * rstd * gb_v[pl.ds(kk * _L, _L)]
                         + gb_v[pl.ds(H + kk * _L, _L)])
                    out_v[pl.ds(off, _L)] = y
                return 0
            lax.fori_loop(0, TPC, tok_body, 0)

            for b in range(B):
                pltpu.sync_copy(
                    out_v.at[pl.ds(b * CS * H, CS * H)],
                    out_hbm.at[pl.ds((b * S + s0) * H, CS * H)])
            return 0

        lax.fori_loop(0, NCH, chunk_body, 0)

    return k


def kernel(input_ids, hash_tables, pos_table, tt_table, gamma, beta):
    B, S = input_ids.shape
    NH, NB, SH = hash_tables.shape
    H = NH * SH
    ids_flat = input_ids.reshape(-1).astype(jnp.int32)
    tab = hash_tables.reshape(NH * NB, SH)
    pos_flat = pos_table.reshape(-1)
    tt0 = tt_table[0]
    out = _make_sc_kernel(B, S, NH, NB, SH)(
        ids_flat, tab, pos_flat, tt0, gamma, beta)
    return out.reshape(B, S, H)
